# Initial kernel scaffold; baseline (speedup 1.0000x reference)
import jax
import jax.numpy as jnp
from jax.experimental import pallas as pl


def kernel(x, enc_w1, enc_b1, enc_w2, enc_b2, enc_w3, enc_b3, dec_w1, dec_b1, dec_w2, dec_b2, dec_w3, dec_b3):
    raise NotImplementedError("write your pallas kernel here")


# speedup vs baseline: 53.5151x; 53.5151x over previous
import jax
import jax.numpy as jnp
from jax.experimental import pallas as pl
from jax.experimental.pallas import tpu as pltpu

_N = 256
_K = 8
_NBR = 16
_NSTEPS = 10
_STEP = 0.01
_REP = 0.001

_pallas_call = pl.pallas_call


def _mlp_kernel(x_ref, w1_ref, b1_ref, w2_ref, b2_ref, w3_ref, b3_ref, o_ref):
    h = jnp.maximum(
        jnp.dot(x_ref[...], w1_ref[...], preferred_element_type=jnp.float32)
        + b1_ref[...], 0.0)
    h = jnp.maximum(
        jnp.dot(h, w2_ref[...], preferred_element_type=jnp.float32)
        + b2_ref[...], 0.0)
    o_ref[...] = jnp.dot(h, w3_ref[...], preferred_element_type=jnp.float32) \
        + b3_ref[...]


def _mlp(x, w1, b1, w2, b2, w3, b3):
    M = x.shape[0]
    return _pallas_call(
        _mlp_kernel,
        out_shape=jax.ShapeDtypeStruct((M, w3.shape[1]), jnp.float32),
    )(x, w1, b1.reshape(1, -1), w2, b2.reshape(1, -1), w3,
      b3.reshape(1, -1))


def _steps_kernel(pos_ref, pref_ref, posout_ref, prefout_ref):
    praw = pos_ref[0]
    prefs = jnp.tanh(pref_ref[0])
    pos0 = jax.nn.sigmoid(praw)

    prods = [prefs[k:k + 1, :].T * prefs[k:k + 1, :] for k in range(_K)]
    b4 = [prods[2 * i] + prods[2 * i + 1] for i in range(4)]
    c2 = [b4[0] + b4[1], b4[2] + b4[3]]
    ip = (c2[0] + c2[1]) / _K

    iota_r = jax.lax.broadcasted_iota(jnp.int32, (_N, _N), 0)
    iota_c = jax.lax.broadcasted_iota(jnp.int32, (_N, _N), 1)
    diag = iota_r == iota_c
    inf = jnp.float32(jnp.inf)

    def one_step(_, pos):
        px = pos[0:1, :]
        py = pos[1:2, :]
        dx = px.T - px
        dx = dx - jnp.round(dx)
        dy = py.T - py
        dy = dy - jnp.round(dy)
        dist = jnp.sqrt(dx * dx + dy * dy)
        d0 = jnp.where(diag, inf, dist)

        dmat = d0
        mterms = []
        pterms = []
        for _r in range(_NBR):
            m = jnp.min(dmat, axis=0, keepdims=True)
            is_min = dmat == m
            idx = jnp.min(jnp.where(is_min, iota_r, jnp.int32(_N)),
                          axis=0, keepdims=True)
            onehot = iota_r == idx
            sel = jnp.where(onehot, 1.0, 0.0)
            tx = jnp.sum(sel * dx, axis=0, keepdims=True)
            ty = jnp.sum(sel * dy, axis=0, keepdims=True)
            ipr = jnp.sum(sel * ip, axis=0, keepdims=True)
            dc = jnp.maximum(m, 1e-12)
            ux = tx / dc
            uy = ty / dc
            dp = jnp.maximum(dc, 1e-6)
            mterms.append((ipr * ux, ipr * uy))
            pterms.append(((-ux) / dp, (-uy) / dp))
            dmat = jnp.where(onehot, inf, dmat)

        def comb(ts):
            b8 = [ts[i] + ts[i + 8] for i in range(8)]
            c4 = [b8[i] + b8[i + 4] for i in range(4)]
            d2_ = [c4[i] + c4[i + 2] for i in range(2)]
            return d2_[0] + d2_[1]

        mx = comb([t[0] for t in mterms])
        my = comb([t[1] for t in mterms])
        qx = comb([t[0] for t in pterms])
        qy = comb([t[1] for t in pterms])

        pxn = px + _STEP * (mx / _NBR) + _REP * (qx / _NBR)
        pyn = py + _STEP * (my / _NBR) + _REP * (qy / _NBR)
        pxn = pxn - jnp.floor(pxn)
        pyn = pyn - jnp.floor(pyn)
        return jnp.concatenate([pxn, pyn], axis=0)

    pos = jax.lax.fori_loop(0, _NSTEPS, one_step, pos0)
    posout_ref[0] = pos
    prefout_ref[0] = prefs


def _run_steps(pos_raw, pref_raw):
    B = pos_raw.shape[0]
    return _pallas_call(
        _steps_kernel,
        grid=(B,),
        in_specs=[
            pl.BlockSpec((1, 2, _N), lambda b: (b, 0, 0)),
            pl.BlockSpec((1, _K, _N), lambda b: (b, 0, 0)),
        ],
        out_specs=[
            pl.BlockSpec((1, 2, _N), lambda b: (b, 0, 0)),
            pl.BlockSpec((1, _K, _N), lambda b: (b, 0, 0)),
        ],
        out_shape=[
            jax.ShapeDtypeStruct((B, 2, _N), jnp.float32),
            jax.ShapeDtypeStruct((B, _K, _N), jnp.float32),
        ],
        compiler_params=pltpu.CompilerParams(
            dimension_semantics=("arbitrary",)),
    )(pos_raw, pref_raw)


def kernel(x, enc_w1, enc_b1, enc_w2, enc_b2, enc_w3, enc_b3,
           dec_w1, dec_b1, dec_w2, dec_b2, dec_w3, dec_b3):
    B = x.shape[0]
    state = _mlp(x, enc_w1, enc_b1, enc_w2, enc_b2, enc_w3, enc_b3)
    pos_raw = state[:, :2 * _N].reshape(B, _N, 2).transpose(0, 2, 1)
    pref_raw = state[:, 2 * _N:].reshape(B, _N, _K).transpose(0, 2, 1)

    pos_f, prefs_f = _run_steps(pos_raw, pref_raw)

    dec_in = jnp.concatenate(
        [pos_f.transpose(0, 2, 1).reshape(B, 2 * _N),
         prefs_f.transpose(0, 2, 1).reshape(B, _K * _N)], axis=1)
    return _mlp(dec_in, dec_w1, dec_b1, dec_w2, dec_b2, dec_w3, dec_b3)
